# baseline (speedup 1.0000x reference)
import functools

import numpy as np
import jax
import jax.numpy as jnp
from jax import lax
from jax.experimental import pallas as pl
from jax.experimental.pallas import tpu as pltpu
from jax.experimental.pallas import tpu_sc as plsc

N_OBS = 1_000_000
N_STATES = 16
BATCH = 4096
HIST = 200

NW = 32
CHUNK = 128
TOTAL = BATCH * HIST
PER_W = TOTAL // NW
NCH = PER_W // CHUNK

NBUCKET = 256
LN2 = float(np.log(2.0))

_i = np.arange(NBUCKET, dtype=np.float64)
_m0 = 1.0 + _i / NBUCKET
_m1 = 1.0 + (_i + 1.0) / NBUCKET
_SLOPE = (np.log(_m1) - np.log(_m0)) / (_m1 - _m0)
_INTERCEPT = np.log(_m0) - _SLOPE * _m0 - 127.0 * np.log(2.0)
_TA = np.asarray(_SLOPE, dtype=np.float32)
_TB = np.asarray(_INTERCEPT, dtype=np.float32)


def _log_rows(rows_ref, slot, outb_ref, ta_ref, tb_ref):
    def row_body(r, carry):
        v = rows_ref[slot, r]
        xi = plsc.bitcast(v, jnp.int32)
        eu = jnp.right_shift(xi, 23)
        bk = jnp.bitwise_and(jnp.right_shift(xi, 15), 255)
        mi = jnp.bitwise_or(jnp.bitwise_and(xi, 0x7FFFFF), 0x3F800000)
        m = plsc.bitcast(mi, jnp.float32)
        a = plsc.load_gather(ta_ref, [bk])
        b = plsc.load_gather(tb_ref, [bk])
        outb_ref[r] = eu.astype(jnp.float32) * LN2 + (a * m + b)
        return carry

    lax.fori_loop(0, CHUNK, row_body, 0)


def _sc_body(idx_hbm, probs_hbm, ta_hbm, tb_hbm, out_hbm,
             idx_v, rows_v, outb_v, ta_v, tb_v, gsem0, gsem1):
    gsems = (gsem0, gsem1)
    wid = lax.axis_index("s") * 2 + lax.axis_index("c")
    base_chunk = wid * NCH

    pltpu.sync_copy(ta_hbm, ta_v)
    pltpu.sync_copy(tb_hbm, tb_v)
    pltpu.sync_copy(idx_hbm.at[pl.ds(base_chunk, NCH)], idx_v)

    pltpu.async_copy(probs_hbm.at[idx_v.at[0]], rows_v.at[0], gsems[0])

    def pair_body(p, carry):
        for s in range(2):
            j = p * 2 + s
            nxt = j + 1

            @pl.when(nxt < NCH)
            def _():
                pltpu.async_copy(probs_hbm.at[idx_v.at[nxt]],
                                 rows_v.at[(s + 1) % 2], gsems[(s + 1) % 2])

            pltpu.make_async_copy(probs_hbm.at[idx_v.at[j]],
                                  rows_v.at[s], gsems[s]).wait()
            _log_rows(rows_v, s, outb_v, ta_v, tb_v)
            pltpu.sync_copy(
                outb_v, out_hbm.at[pl.ds((base_chunk + j) * CHUNK, CHUNK)])
        return carry

    lax.fori_loop(0, NCH // 2, pair_body, 0)


@jax.jit
def kernel(x, probs):
    idx2d = x.reshape(TOTAL // CHUNK, CHUNK).astype(jnp.int32)
    mesh = plsc.VectorSubcoreMesh(core_axis_name="c", subcore_axis_name="s")
    out = pl.kernel(
        _sc_body,
        out_type=jax.ShapeDtypeStruct((TOTAL, N_STATES), jnp.float32),
        mesh=mesh,
        compiler_params=pltpu.CompilerParams(
            needs_layout_passes=False, use_tc_tiling_on_sc=False),
        scratch_types=[
            pltpu.VMEM((NCH, CHUNK), jnp.int32),
            pltpu.VMEM((2, CHUNK, N_STATES), jnp.float32),
            pltpu.VMEM((CHUNK, N_STATES), jnp.float32),
            pltpu.VMEM((NBUCKET,), jnp.float32),
            pltpu.VMEM((NBUCKET,), jnp.float32),
            pltpu.SemaphoreType.DMA,
            pltpu.SemaphoreType.DMA,
        ],
    )(idx2d, probs, jnp.asarray(_TA), jnp.asarray(_TB))
    return out.reshape(BATCH, HIST, N_STATES)
